# baseline (speedup 1.0000x reference)
import dataclasses
import functools

import jax
import jax.numpy as jnp
from jax import lax
from jax.experimental import pallas as pl
from jax.experimental.pallas import tpu as pltpu
from jax.experimental.pallas import tpu_sc as plsc

DIM = 32
NUM_WORKERS = 32
CHUNK = 320
SUB = 160
SUBS_PER_CHUNK = CHUNK // SUB
GROUP = 16


def _sc_norm_sum(W, cidx, pidx):
    n = cidx.shape[0]
    n_full = n // CHUNK
    tail = n - n_full * CHUNK
    assert tail % GROUP == 0 and tail % 8 == 0 and tail <= CHUNK
    nc = -(-n_full // NUM_WORKERS)
    assert nc >= 3

    mesh = plsc.VectorSubcoreMesh(core_axis_name="c", subcore_axis_name="s")
    cp = pltpu.CompilerParams()
    if "needs_layout_passes" in pltpu.CompilerParams.__dataclass_fields__:
        cp = dataclasses.replace(cp, needs_layout_passes=False)
    if "use_tc_tiling_on_sc" in pltpu.CompilerParams.__dataclass_fields__:
        cp = dataclasses.replace(cp, use_tc_tiling_on_sc=False)

    @functools.partial(
        pl.kernel,
        compiler_params=cp,
        out_type=jax.ShapeDtypeStruct((NUM_WORKERS, GROUP), jnp.float32),
        mesh=mesh,
        scratch_types=(
            [pltpu.VMEM((CHUNK,), jnp.int32)] * 6
            + [pltpu.VMEM((CHUNK, DIM), jnp.float32)] * 6
            + [pltpu.VMEM((GROUP,), jnp.float32)]
            + [pltpu.SemaphoreType.DMA] * 6
        ),
    )
    def k(w_hbm, ci_hbm, pi_hbm, out_hbm,
          ci0, ci1, ci2, pi0, pi1, pi2,
          crows0, crows1, crows2, prows0, prows1, prows2, acc,
          isem0, isem1, isem2, gsem0, gsem1, gsem2):
        wid = lax.axis_index("s") * 2 + lax.axis_index("c")
        acc[...] = jnp.zeros((GROUP,), jnp.float32)
        lane = lax.iota(jnp.int32, GROUP)
        lo8 = lane < 8
        m8 = lane & 7
        x8 = lane ^ 8
        ff2 = (m8 & 3) | ((m8 >> 2) << 3)
        gg2 = ff2 + 4
        ff3 = (m8 & 1) | ((m8 >> 1) << 2)
        gg3 = ff3 + 2
        ff4 = m8 << 1
        gg4 = ff4 + 1

        gd = lax.GatherDimensionNumbers(
            offset_dims=(), collapsed_slice_dims=(0,), start_index_map=(0,))

        def take(v, idx):
            return lax.gather(v, idx[:, None], gd, slice_sizes=(1,),
                              mode=lax.GatherScatterMode.PROMISE_IN_BOUNDS)

        def comb1(a, b):
            return (jnp.where(lo8, a, take(b, x8))
                    + jnp.where(lo8, take(a, x8), b))

        def comb(a, b, ff, gg):
            return (jnp.where(lo8, take(a, ff), take(b, ff))
                    + jnp.where(lo8, take(a, gg), take(b, gg)))

        def base_of(kk):
            cg = wid + NUM_WORKERS * kk
            return jnp.where(cg < n_full, cg, 0) * CHUNK


        def ifetch(kk, civ, piv, isem):
            b = base_of(kk)
            pltpu.async_copy(ci_hbm.at[pl.ds(b, CHUNK)], civ, isem)
            pltpu.async_copy(pi_hbm.at[pl.ds(b, CHUNK)], piv, isem)

        def iwait(civ, piv, isem):
            pltpu.make_async_copy(ci_hbm.at[pl.ds(0, CHUNK)], civ, isem).wait()
            pltpu.make_async_copy(pi_hbm.at[pl.ds(0, CHUNK)], piv, isem).wait()

        def gissue(civ, piv, cr, pr, gsem):
            for j in range(SUBS_PER_CHUNK):
                sl = pl.ds(j * SUB, SUB)
                pltpu.async_copy(w_hbm.at[civ.at[sl]], cr.at[sl], gsem)
                pltpu.async_copy(w_hbm.at[piv.at[sl]], pr.at[sl], gsem)

        def gwait(cr, pr, gsem):
            dummy = w_hbm.at[pl.ds(0, CHUNK)]
            pltpu.make_async_copy(dummy, cr, gsem).wait()
            pltpu.make_async_copy(dummy, pr, gsem).wait()

        def group_sqnorms(cr, pr, g):
            base = g * GROUP
            ss = []
            for kk in range(GROUP):
                r = base + kk
                d0 = cr[r, pl.ds(0, 16)] - pr[r, pl.ds(0, 16)]
                d1 = cr[r, pl.ds(16, 16)] - pr[r, pl.ds(16, 16)]
                ss.append(d0 * d0 + d1 * d1)
            t = [comb1(ss[2 * j], ss[2 * j + 1]) for j in range(8)]
            u = [comb(t[2 * j], t[2 * j + 1], ff2, gg2) for j in range(4)]
            w = [comb(u[2 * j], u[2 * j + 1], ff3, gg3) for j in range(2)]
            return comb(w[0], w[1], ff4, gg4)

        def vsqrt(s):
            sc = jnp.maximum(s, jnp.float32(1e-30))
            i = plsc.bitcast(sc, jnp.int32)
            i = jnp.int32(0x5F3759DF) - lax.shift_right_logical(i, 1)
            y = plsc.bitcast(i, jnp.float32)
            y = y * (jnp.float32(1.5) - jnp.float32(0.5) * sc * y * y)
            return sc * y

        def compute(cr, pr, ngroups):
            assert ngroups % 4 == 0 or ngroups < 4

            if ngroups >= 4:
                @pl.loop(0, ngroups, step=4)
                def _group(g):
                    total = acc[...]
                    for u in range(4):
                        s = group_sqnorms(cr, pr, g + u)
                        total = total + vsqrt(s)
                    acc[...] = total
            else:
                @pl.loop(0, ngroups)
                def _group(g):
                    s = group_sqnorms(cr, pr, g)
                    acc[...] = acc[...] + vsqrt(s)

        ci = (ci0, ci1, ci2)
        pi = (pi0, pi1, pi2)
        cr = (crows0, crows1, crows2)
        pr = (prows0, prows1, prows2)
        isem = (isem0, isem1, isem2)
        gsem = (gsem0, gsem1, gsem2)

        ifetch(0, ci[0], pi[0], isem[0])
        ifetch(1, ci[1], pi[1], isem[1])
        ifetch(2, ci[2], pi[2], isem[2])
        iwait(ci[0], pi[0], isem[0])
        gissue(ci[0], pi[0], cr[0], pr[0], gsem[0])
        iwait(ci[1], pi[1], isem[1])
        gissue(ci[1], pi[1], cr[1], pr[1], gsem[1])

        @pl.loop(0, 3 * ((nc + 2) // 3), step=3)
        def _pipe(kk):
            for i in range(3):
                s = i
                s2 = (i + 2) % 3

                @pl.when(kk + i < nc)
                def _(i=i, s=s, s2=s2):
                    gwait(cr[s], pr[s], gsem[s])

                    @pl.when(kk + i + 3 < nc)
                    def _():
                        ifetch(kk + i + 3, ci[s], pi[s], isem[s])

                    @pl.when(kk + i + 2 < nc)
                    def _():
                        iwait(ci[s2], pi[s2], isem[s2])
                        gissue(ci[s2], pi[s2], cr[s2], pr[s2], gsem[s2])

                    @pl.when(wid + NUM_WORKERS * (kk + i) < n_full)
                    def _():
                        compute(cr[s], pr[s], CHUNK // GROUP)

        if tail:
            @pl.when(wid == 0)
            def _tail():
                tb = n_full * CHUNK
                pltpu.sync_copy(ci_hbm.at[pl.ds(tb, tail)],
                                ci0.at[pl.ds(0, tail)])
                pltpu.sync_copy(pi_hbm.at[pl.ds(tb, tail)],
                                pi0.at[pl.ds(0, tail)])
                pltpu.sync_copy(w_hbm.at[ci0.at[pl.ds(0, tail)]],
                                crows0.at[pl.ds(0, tail)])
                pltpu.sync_copy(w_hbm.at[pi0.at[pl.ds(0, tail)]],
                                prows0.at[pl.ds(0, tail)])
                compute(crows0, prows0, tail // GROUP)

        pltpu.sync_copy(acc, out_hbm.at[wid])

    return k(W, cidx, pidx)


def kernel(W, childs_idx, parents_idx):
    partials = _sc_norm_sum(W, childs_idx, parents_idx)
    return jnp.sum(partials) / 2.0


# speedup vs baseline: 1.5006x; 1.5006x over previous
import dataclasses
import functools

import jax
import jax.numpy as jnp
from jax import lax
from jax.experimental import pallas as pl
from jax.experimental.pallas import tpu as pltpu
from jax.experimental.pallas import tpu_sc as plsc

DIM = 32
NUM_WORKERS = 32
CHUNK = 320
SUB = 160
SUBS_PER_CHUNK = CHUNK // SUB
GROUP = 16


def _sc_norm_sum(W, cidx, pidx):
    n = cidx.shape[0]
    n_full = n // CHUNK
    tail = n - n_full * CHUNK
    assert tail % GROUP == 0 and tail % 8 == 0 and tail <= CHUNK
    nc = -(-n_full // NUM_WORKERS)
    assert nc >= 3

    mesh = plsc.VectorSubcoreMesh(core_axis_name="c", subcore_axis_name="s")
    cp = pltpu.CompilerParams()
    if "needs_layout_passes" in pltpu.CompilerParams.__dataclass_fields__:
        cp = dataclasses.replace(cp, needs_layout_passes=False)
    if "use_tc_tiling_on_sc" in pltpu.CompilerParams.__dataclass_fields__:
        cp = dataclasses.replace(cp, use_tc_tiling_on_sc=False)

    @functools.partial(
        pl.kernel,
        compiler_params=cp,
        out_type=jax.ShapeDtypeStruct((NUM_WORKERS, GROUP), jnp.float32),
        mesh=mesh,
        scratch_types=(
            [pltpu.VMEM((CHUNK,), jnp.int32)] * 6
            + [pltpu.VMEM((CHUNK, DIM), jnp.float32)] * 6
            + [pltpu.VMEM((GROUP,), jnp.float32)]
            + [pltpu.SemaphoreType.DMA] * 6
        ),
    )
    def k(w_hbm, ci_hbm, pi_hbm, out_hbm,
          ci0, ci1, ci2, pi0, pi1, pi2,
          crows0, crows1, crows2, prows0, prows1, prows2, acc,
          isem0, isem1, isem2, gsem0, gsem1, gsem2):
        wid = lax.axis_index("s") * 2 + lax.axis_index("c")
        acc[...] = jnp.zeros((GROUP,), jnp.float32)
        lane = lax.iota(jnp.int32, GROUP)
        lo8 = lane < 8
        m8 = lane & 7
        x8 = lane ^ 8
        ff2 = (m8 & 3) | ((m8 >> 2) << 3)
        gg2 = ff2 + 4
        ff3 = (m8 & 1) | ((m8 >> 1) << 2)
        gg3 = ff3 + 2
        ff4 = m8 << 1
        gg4 = ff4 + 1

        gd = lax.GatherDimensionNumbers(
            offset_dims=(), collapsed_slice_dims=(0,), start_index_map=(0,))

        def take(v, idx):
            return lax.gather(v, idx[:, None], gd, slice_sizes=(1,),
                              mode=lax.GatherScatterMode.PROMISE_IN_BOUNDS)

        def comb1(a, b):
            return (jnp.where(lo8, a, take(b, x8))
                    + jnp.where(lo8, take(a, x8), b))

        def comb(a, b, ff, gg):
            return (jnp.where(lo8, take(a, ff), take(b, ff))
                    + jnp.where(lo8, take(a, gg), take(b, gg)))

        def base_of(kk):
            cg = wid + NUM_WORKERS * kk
            return jnp.where(cg < n_full, cg, 0) * CHUNK


        def ifetch(kk, civ, piv, isem):
            b = base_of(kk)
            pltpu.async_copy(ci_hbm.at[pl.ds(b, CHUNK)], civ, isem)
            pltpu.async_copy(pi_hbm.at[pl.ds(b, CHUNK)], piv, isem)

        def iwait(civ, piv, isem):
            pltpu.make_async_copy(ci_hbm.at[pl.ds(0, CHUNK)], civ, isem).wait()
            pltpu.make_async_copy(pi_hbm.at[pl.ds(0, CHUNK)], piv, isem).wait()

        def gissue(civ, piv, cr, pr, gsem):
            for j in range(SUBS_PER_CHUNK):
                sl = pl.ds(j * SUB, SUB)
                pltpu.async_copy(w_hbm.at[civ.at[sl]], cr.at[sl], gsem)
                pltpu.async_copy(w_hbm.at[piv.at[sl]], pr.at[sl], gsem)

        def gwait(cr, pr, gsem):
            dummy = w_hbm.at[pl.ds(0, CHUNK)]
            pltpu.make_async_copy(dummy, cr, gsem).wait()
            pltpu.make_async_copy(dummy, pr, gsem).wait()

        def group_sqnorms(cr, pr, g):
            base = g * GROUP
            ss = []
            for kk in range(GROUP):
                r = base + kk
                d0 = cr[r, pl.ds(0, 16)] - pr[r, pl.ds(0, 16)]
                d1 = cr[r, pl.ds(16, 16)] - pr[r, pl.ds(16, 16)]
                ss.append(d0 * d0 + d1 * d1)
            t = [comb1(ss[2 * j], ss[2 * j + 1]) for j in range(8)]
            u = [comb(t[2 * j], t[2 * j + 1], ff2, gg2) for j in range(4)]
            w = [comb(u[2 * j], u[2 * j + 1], ff3, gg3) for j in range(2)]
            return comb(w[0], w[1], ff4, gg4)

        def vsqrt(s):
            sc = jnp.maximum(s, jnp.float32(1e-30))
            i = plsc.bitcast(sc, jnp.int32)
            i = jnp.int32(0x5F3759DF) - lax.shift_right_logical(i, 1)
            y = plsc.bitcast(i, jnp.float32)
            y = y * (jnp.float32(1.5) - jnp.float32(0.5) * sc * y * y)
            return sc * y

        def compute(cr, pr, ngroups):
            assert ngroups % 2 == 0 or ngroups < 2

            if ngroups >= 2:
                @pl.loop(0, ngroups, step=2)
                def _group(g):
                    s0 = group_sqnorms(cr, pr, g)
                    t0 = vsqrt(s0)
                    s1 = group_sqnorms(cr, pr, g + 1)
                    acc[...] = acc[...] + (t0 + vsqrt(s1))
            else:
                @pl.loop(0, ngroups)
                def _group(g):
                    s = group_sqnorms(cr, pr, g)
                    acc[...] = acc[...] + vsqrt(s)

        ci = (ci0, ci1, ci2)
        pi = (pi0, pi1, pi2)
        cr = (crows0, crows1, crows2)
        pr = (prows0, prows1, prows2)
        isem = (isem0, isem1, isem2)
        gsem = (gsem0, gsem1, gsem2)

        ifetch(0, ci[0], pi[0], isem[0])
        ifetch(1, ci[1], pi[1], isem[1])
        ifetch(2, ci[2], pi[2], isem[2])
        iwait(ci[0], pi[0], isem[0])
        gissue(ci[0], pi[0], cr[0], pr[0], gsem[0])
        iwait(ci[1], pi[1], isem[1])
        gissue(ci[1], pi[1], cr[1], pr[1], gsem[1])

        @pl.loop(0, 3 * ((nc + 2) // 3), step=3)
        def _pipe(kk):
            for i in range(3):
                s = i
                s2 = (i + 2) % 3

                @pl.when(kk + i < nc)
                def _(i=i, s=s, s2=s2):
                    gwait(cr[s], pr[s], gsem[s])

                    @pl.when(kk + i + 3 < nc)
                    def _():
                        ifetch(kk + i + 3, ci[s], pi[s], isem[s])

                    @pl.when(kk + i + 2 < nc)
                    def _():
                        iwait(ci[s2], pi[s2], isem[s2])
                        gissue(ci[s2], pi[s2], cr[s2], pr[s2], gsem[s2])

                    @pl.when(wid + NUM_WORKERS * (kk + i) < n_full)
                    def _():
                        compute(cr[s], pr[s], CHUNK // GROUP)

        if tail:
            @pl.when(wid == 0)
            def _tail():
                tb = n_full * CHUNK
                pltpu.sync_copy(ci_hbm.at[pl.ds(tb, tail)],
                                ci0.at[pl.ds(0, tail)])
                pltpu.sync_copy(pi_hbm.at[pl.ds(tb, tail)],
                                pi0.at[pl.ds(0, tail)])
                pltpu.sync_copy(w_hbm.at[ci0.at[pl.ds(0, tail)]],
                                crows0.at[pl.ds(0, tail)])
                pltpu.sync_copy(w_hbm.at[pi0.at[pl.ds(0, tail)]],
                                prows0.at[pl.ds(0, tail)])
                compute(crows0, prows0, tail // GROUP)

        pltpu.sync_copy(acc, out_hbm.at[wid])

    return k(W, cidx, pidx)


def kernel(W, childs_idx, parents_idx):
    partials = _sc_norm_sum(W, childs_idx, parents_idx)
    return jnp.sum(partials) / 2.0
